# baseline (speedup 1.0000x reference)
import functools

import jax
import jax.numpy as jnp
from jax import lax
from jax.experimental import pallas as pl
from jax.experimental.pallas import tpu as pltpu
from jax.experimental.pallas import tpu_sc as plsc

NC = 2
NS = 16
L = 16
NW = NC * NS
B = 16384
K = 32
BPW = B // NW
CHUNK = 128
NCHUNK = BPW // CHUNK
NBLK = BPW // L


def _bit_reverse4(j: int) -> int:
    return ((j & 1) << 3) | ((j & 2) << 1) | ((j & 4) >> 1) | ((j & 8) >> 3)


_mesh = plsc.VectorSubcoreMesh(core_axis_name="c", subcore_axis_name="s",
                               num_cores=NC, num_subcores=NS)


@functools.partial(
    pl.kernel,
    out_type=jax.ShapeDtypeStruct((B,), jnp.float32),
    mesh=_mesh,
    scratch_types=[
        pltpu.VMEM((NCHUNK, CHUNK), jnp.int32),
        pltpu.VMEM((NCHUNK, CHUNK), jnp.int32),
        pltpu.VMEM((BPW, K), jnp.float32),
        pltpu.VMEM((BPW, K), jnp.float32),
        pltpu.VMEM((BPW,), jnp.float32),
        pltpu.SemaphoreType.DMA,
        pltpu.SemaphoreType.DMA,
    ],
    compiler_params=pltpu.CompilerParams(use_tc_tiling_on_sc=False),
)
def _mf_sc(user_hbm, item_hbm, uf_hbm, vf_hbm, out_hbm,
           uidx, iidx, urows, vrows, outv, sem_u, sem_v):
    wid = lax.axis_index("s") * NC + lax.axis_index("c")
    base = wid * BPW

    pltpu.sync_copy(user_hbm.at[wid], uidx)
    pltpu.sync_copy(item_hbm.at[wid], iidx)

    copies = []
    for j in range(NCHUNK):
        dst = pl.ds(j * CHUNK, CHUNK)
        copies.append(pltpu.async_copy(uf_hbm.at[uidx.at[j]], urows.at[dst], sem_u))
        copies.append(pltpu.async_copy(vf_hbm.at[iidx.at[j]], vrows.at[dst], sem_v))
    for c in copies:
        c.wait()

    lane = lax.iota(jnp.int32, L)
    masks = [(lane & d) == 0 for d in (8, 4, 2, 1)]
    perms = [lane ^ d for d in (8, 4, 2, 1)]
    brev = (((lane & 1) << 3) | ((lane & 2) << 1)
            | ((lane & 4) >> 1) | ((lane & 8) >> 3))

    def block(i, carry):
        rowbase = i * L
        vecs = []
        for r in range(L):
            row = rowbase + r
            u0 = urows[row, pl.ds(0, L)]
            u1 = urows[row, pl.ds(L, L)]
            v0 = vrows[row, pl.ds(0, L)]
            v1 = vrows[row, pl.ds(L, L)]
            vecs.append(u0 * v0 + u1 * v1)
        for lvl in range(4):
            m, pm = masks[lvl], perms[lvl]
            nxt = []
            for a_i in range(0, len(vecs), 2):
                a, b = vecs[a_i], vecs[a_i + 1]
                sa = a.at[pm].get(mode="promise_in_bounds")
                sb = b.at[pm].get(mode="promise_in_bounds")
                nxt.append(jnp.where(m, a, sb) + jnp.where(m, sa, b))
            vecs = nxt
        z = vecs[0].at[brev].get(mode="promise_in_bounds")
        outv[pl.ds(rowbase, L)] = z
        return carry

    lax.fori_loop(0, NBLK, block, 0)

    pltpu.sync_copy(outv, out_hbm.at[pl.ds(base, BPW)])


def kernel(user, item, user_factors, item_factors):
    user_r = user.reshape(NW, NCHUNK, CHUNK)
    item_r = item.reshape(NW, NCHUNK, CHUNK)
    return _mf_sc(user_r, item_r, user_factors, item_factors)


# speedup vs baseline: 3.1003x; 3.1003x over previous
import functools

import jax
import jax.numpy as jnp
from jax import lax
from jax.experimental import pallas as pl
from jax.experimental.pallas import tpu as pltpu
from jax.experimental.pallas import tpu_sc as plsc

NC = 2
NS = 16
L = 16
NW = NC * NS
B = 16384
K = 32
BPW = B // NW
GB = 4
NGRP = BPW // GB
NBLK = BPW // L
W = 128

_mesh = plsc.VectorSubcoreMesh(core_axis_name="c", subcore_axis_name="s",
                               num_cores=NC, num_subcores=NS)


@functools.partial(
    pl.kernel,
    out_type=jax.ShapeDtypeStruct((B,), jnp.float32),
    mesh=_mesh,
    scratch_types=[
        pltpu.VMEM((BPW,), jnp.int32),
        pltpu.VMEM((BPW,), jnp.int32),
        pltpu.VMEM((K,), jnp.int32),
        pltpu.VMEM((2, GB, K, W), jnp.float32),
        pltpu.VMEM((2, GB, K, W), jnp.float32),
        pltpu.VMEM((BPW,), jnp.float32),
        pltpu.SemaphoreType.DMA,
        pltpu.SemaphoreType.DMA,
    ],
    compiler_params=pltpu.CompilerParams(use_tc_tiling_on_sc=True,
                                         needs_layout_passes=False),
)
def _mf_sc(user_hbm, item_hbm, uft, ift, out_hbm,
           uidx, iidx, kidx, ubuf, vbuf, outv, sem_u, sem_v):
    wid = lax.axis_index("s") * NC + lax.axis_index("c")
    base = wid * BPW

    pltpu.sync_copy(user_hbm.at[pl.ds(base, BPW)], uidx)
    pltpu.sync_copy(item_hbm.at[pl.ds(base, BPW)], iidx)

    lane = lax.iota(jnp.int32, L)
    kidx[pl.ds(0, L)] = lane
    kidx[pl.ds(L, L)] = lane + L

    def fire(iu_vec, ii_vec, sub, g):
        slot = lax.rem(g, 2)
        for j in range(GB):
            cu = (iu_vec[sub * GB + j] // W) * W
            ci = (ii_vec[sub * GB + j] // W) * W
            pltpu.async_copy(uft.at[kidx, pl.ds(cu, W)],
                             ubuf.at[slot, j], sem_u)
            pltpu.async_copy(ift.at[kidx, pl.ds(ci, W)],
                             vbuf.at[slot, j], sem_v)

    def drain(g):
        slot = lax.rem(g, 2)
        for j in range(GB):
            pltpu.make_async_copy(uft.at[kidx, pl.ds(0, W)],
                                  ubuf.at[slot, j], sem_u).wait()
            pltpu.make_async_copy(ift.at[kidx, pl.ds(0, W)],
                                  vbuf.at[slot, j], sem_v).wait()

    masks = [(lane & d) == 0 for d in (8, 4, 2, 1)]
    perms = [lane ^ d for d in (8, 4, 2, 1)]
    brev = (((lane & 1) << 3) | ((lane & 2) << 1)
            | ((lane & 4) >> 1) | ((lane & 8) >> 3))

    iu0 = uidx[pl.ds(0, L)]
    ii0 = iidx[pl.ds(0, L)]
    fire(iu0, ii0, 0, jnp.int32(0))

    def block(blk, carry):
        iu_vec = uidx[pl.ds(blk * L, L)]
        ii_vec = iidx[pl.ds(blk * L, L)]
        ps = []
        for sub in range(4):
            g = blk * 4 + sub
            drain(g)
            if sub < 3:
                fire(iu_vec, ii_vec, sub + 1, g + 1)
            else:
                @pl.when(blk + 1 < NBLK)
                def _():
                    iu_n = uidx[pl.ds((blk + 1) * L, L)]
                    ii_n = iidx[pl.ds((blk + 1) * L, L)]
                    fire(iu_n, ii_n, 0, g + 1)

            slot = lax.rem(g, 2)
            for j in range(GB):
                colu = lax.rem(iu_vec[sub * GB + j], W)
                coli = lax.rem(ii_vec[sub * GB + j], W)
                cu = jnp.full((L,), colu, jnp.int32)
                ci = jnp.full((L,), coli, jnp.int32)
                u0 = plsc.load_gather(ubuf.at[slot, j], [lane, cu])
                u1 = plsc.load_gather(ubuf.at[slot, j], [lane + L, cu])
                v0 = plsc.load_gather(vbuf.at[slot, j], [lane, ci])
                v1 = plsc.load_gather(vbuf.at[slot, j], [lane + L, ci])
                ps.append(u0 * v0 + u1 * v1)
        vecs = ps
        for lvl in range(4):
            m, pm = masks[lvl], perms[lvl]
            nxt = []
            for a_i in range(0, len(vecs), 2):
                a, bb = vecs[a_i], vecs[a_i + 1]
                sa = a.at[pm].get(mode="promise_in_bounds")
                sb = bb.at[pm].get(mode="promise_in_bounds")
                nxt.append(jnp.where(m, a, sb) + jnp.where(m, sa, bb))
            vecs = nxt
        z = vecs[0].at[brev].get(mode="promise_in_bounds")
        outv[pl.ds(blk * L, L)] = z
        return carry

    lax.fori_loop(0, NBLK, block, 0)

    pltpu.sync_copy(outv, out_hbm.at[pl.ds(base, BPW)])


def kernel(user, item, user_factors, item_factors):
    return _mf_sc(user, item, user_factors.T, item_factors.T)
